# baseline (speedup 1.0000x reference)
import functools

import jax
import jax.numpy as jnp
from jax import lax
from jax.experimental import pallas as pl
from jax.experimental.pallas import tpu as pltpu
from jax.experimental.pallas import tpu_sc as plsc

NC = 2
NS = 16
NW = NC * NS
CHUNK = 128


def _sc_mesh():
    return plsc.VectorSubcoreMesh(
        core_axis_name="c", subcore_axis_name="s", num_cores=NC, num_subcores=NS
    )


def _make_degree_call(acc_n, ch):
    rps = acc_n // NS

    def body(dst_hbm, zeros_hbm, out_hbm, idx_d, ones_v, degsh, sem):
        cid = lax.axis_index("c")
        sid = lax.axis_index("s")
        wid = sid * NC + cid
        pltpu.sync_copy(
            zeros_hbm.at[pl.ds(sid * rps, rps)], degsh.at[pl.ds(sid * rps, rps)]
        )
        pltpu.sync_copy(dst_hbm.at[wid], idx_d)
        for t in range(CHUNK // 16):
            ones_v[pl.ds(t * 16, 16)] = jnp.ones((16,), jnp.float32)
        plsc.subcore_barrier()

        def step(j, c):
            pltpu.sync_copy(ones_v, degsh.at[idx_d.at[j]], add=True)
            return c

        lax.fori_loop(0, ch, step, 0)
        plsc.subcore_barrier()
        pltpu.sync_copy(
            degsh.at[pl.ds(sid * rps, rps)],
            out_hbm.at[cid].at[pl.ds(sid * rps, rps)],
        )

    return pl.kernel(
        body,
        out_type=jax.ShapeDtypeStruct((NC, acc_n), jnp.float32),
        mesh=_sc_mesh(),
        scratch_types=[
            pltpu.VMEM((ch, CHUNK), jnp.int32),
            pltpu.VMEM((CHUNK,), jnp.float32),
            pltpu.VMEM_SHARED((acc_n,), jnp.float32),
            pltpu.SemaphoreType.DMA,
        ],
    )


def _make_scatter_call(acc_n, ch, h):
    rps = acc_n // NS

    assert ch % 4 == 0
    sch = ch // 2

    def body(feat_hbm, src_hbm, dst_hbm, zeros_hbm, out_hbm,
             idx_s, idx_d, rows0, rows1, accsh, gsem0, gsem1, ssem0, ssem1):
        cid = lax.axis_index("c")
        sid = lax.axis_index("s")
        wid = sid * NC + cid
        pltpu.sync_copy(
            zeros_hbm.at[pl.ds(sid * rps, rps)], accsh.at[pl.ds(sid * rps, rps)]
        )
        plsc.subcore_barrier()

        for phase in range(2):
            base = phase * sch
            pltpu.sync_copy(src_hbm.at[wid].at[pl.ds(base, sch)], idx_s)
            pltpu.sync_copy(dst_hbm.at[wid].at[pl.ds(base, sch)], idx_d)
            pltpu.async_copy(feat_hbm.at[idx_s.at[0]], rows0, gsem0)
            pltpu.async_copy(feat_hbm.at[idx_s.at[1]], rows1, gsem1)

            def step(i, c):
                j0 = 2 * i
                j1 = 2 * i + 1
                pltpu.make_async_copy(
                    feat_hbm.at[idx_s.at[j0]], rows0, gsem0
                ).wait()
                pltpu.async_copy(rows0, accsh.at[idx_d.at[j0]], ssem0, add=True)
                pltpu.make_async_copy(
                    feat_hbm.at[idx_s.at[j1]], rows1, gsem1
                ).wait()
                pltpu.async_copy(rows1, accsh.at[idx_d.at[j1]], ssem1, add=True)

                @pl.when(j0 + 2 < sch)
                def _():
                    pltpu.make_async_copy(
                        rows0, accsh.at[idx_d.at[j0]], ssem0
                    ).wait()
                    pltpu.async_copy(feat_hbm.at[idx_s.at[j0 + 2]], rows0, gsem0)

                @pl.when(j1 + 2 < sch)
                def _():
                    pltpu.make_async_copy(
                        rows1, accsh.at[idx_d.at[j1]], ssem1
                    ).wait()
                    pltpu.async_copy(feat_hbm.at[idx_s.at[j1 + 2]], rows1, gsem1)

                return c

            lax.fori_loop(0, sch // 2, step, 0)
            pltpu.make_async_copy(rows0, accsh.at[idx_d.at[sch - 2]], ssem0).wait()
            pltpu.make_async_copy(rows1, accsh.at[idx_d.at[sch - 1]], ssem1).wait()
        plsc.subcore_barrier()
        pltpu.sync_copy(
            accsh.at[pl.ds(sid * rps, rps)],
            out_hbm.at[cid].at[pl.ds(sid * rps, rps)],
        )

    return pl.kernel(
        body,
        out_type=jax.ShapeDtypeStruct((NC, acc_n, h), jnp.float32),
        mesh=_sc_mesh(),
        scratch_types=[
            pltpu.VMEM((sch, CHUNK), jnp.int32),
            pltpu.VMEM((sch, CHUNK), jnp.int32),
            pltpu.VMEM((CHUNK, h), jnp.float32),
            pltpu.VMEM((CHUNK, h), jnp.float32),
            pltpu.VMEM_SHARED((acc_n, h), jnp.float32),
            pltpu.SemaphoreType.DMA,
            pltpu.SemaphoreType.DMA,
            pltpu.SemaphoreType.DMA,
            pltpu.SemaphoreType.DMA,
        ],
    )


def _tc_pre(x_ref, w_ref, degp_ref, out_ref):
    dis = lax.rsqrt(degp_ref[0, :] + degp_ref[1, :] + 1.0)
    hval = jnp.dot(x_ref[...], w_ref[...], preferred_element_type=jnp.float32)
    out_ref[...] = (hval * dis[:, None]).astype(out_ref.dtype)


def _tc_mid(accp_ref, hs_ref, degp_ref, b_ref, w_ref, out_ref):
    dis = lax.rsqrt(degp_ref[0, :] + degp_ref[1, :] + 1.0)
    acc = (accp_ref[0].astype(jnp.float32) + accp_ref[1].astype(jnp.float32)
           + hs_ref[...].astype(jnp.float32))
    x1 = jnp.maximum(acc * dis[:, None] + b_ref[...], 0.0)
    h2 = jnp.dot(x1, w_ref[...], preferred_element_type=jnp.float32)
    out_ref[...] = (h2 * dis[:, None]).astype(out_ref.dtype)


def _tc_final(accp_ref, hs_ref, degp_ref, b_ref, w_ref, blin_ref, out_ref):
    dis = lax.rsqrt(degp_ref[0, :] + degp_ref[1, :] + 1.0)
    acc = (accp_ref[0].astype(jnp.float32) + accp_ref[1].astype(jnp.float32)
           + hs_ref[...].astype(jnp.float32))
    x2 = jnp.maximum(acc * dis[:, None] + b_ref[...], 0.0)
    logits = jnp.dot(x2, w_ref[...], preferred_element_type=jnp.float32)
    logits = logits + blin_ref[...]
    m = jnp.max(logits, axis=1, keepdims=True)
    s = logits - m
    lse = jnp.log(jnp.sum(jnp.exp(s), axis=1, keepdims=True))
    out_ref[...] = s - lse


def kernel(x, edge_index, W1, b1, W2, b2, Wlin, blin):
    n, f = x.shape
    h = W1.shape[1]
    c = Wlin.shape[1]
    e = edge_index.shape[1]

    acc_n = -(-(n + 1) // (NS * CHUNK)) * (NS * CHUNK)
    dr = acc_n - n

    idt = edge_index.dtype
    if e % NW == 0:
        epw = e // NW
        ch = -(-epw // CHUNK)
        ch = -(-ch // 4) * 4
        pad = ch * CHUNK - epw
        w_ids = jnp.arange(NW, dtype=idt)[:, None]
        k_ids = jnp.arange(pad, dtype=idt)[None, :]
        pad_dst = (n + (k_ids + w_ids * 8) % dr).astype(idt)
        pad_src = ((k_ids * 61 + w_ids * 7) % n).astype(idt)
        src_p = jnp.concatenate(
            [edge_index[0].reshape(NW, epw), pad_src], axis=1
        ).reshape(NW, ch, CHUNK)
        dst_p = jnp.concatenate(
            [edge_index[1].reshape(NW, epw), pad_dst], axis=1
        ).reshape(NW, ch, CHUNK)
    else:
        ch = -(-e // (NW * CHUNK))
        ch = -(-ch // 4) * 4
        pad = NW * ch * CHUNK - e
        k_ids = jnp.arange(pad, dtype=idt)
        src_p = jnp.concatenate(
            [edge_index[0], (k_ids * 61) % n]
        ).reshape(NW, ch, CHUNK)
        dst_p = jnp.concatenate(
            [edge_index[1], n + k_ids % dr]
        ).reshape(NW, ch, CHUNK)
    fdt = jnp.float32
    zeros2 = jnp.zeros((acc_n, h), fdt)
    zeros1 = jnp.zeros((acc_n,), jnp.float32)

    degp = _make_degree_call(acc_n, ch)(dst_p, zeros1)
    scatter = _make_scatter_call(acc_n, ch, h)

    br = 1280
    grid = (acc_n // br,)
    mm = functools.partial(
        pl.pallas_call,
        grid=grid,
    )

    hs1 = mm(
        _tc_pre,
        in_specs=[
            pl.BlockSpec((br, f), lambda i: (i, 0)),
            pl.BlockSpec((f, h), lambda i: (0, 0)),
            pl.BlockSpec((NC, br), lambda i: (0, i)),
        ],
        out_specs=pl.BlockSpec((br, h), lambda i: (i, 0)),
        out_shape=jax.ShapeDtypeStruct((n, h), fdt),
    )(x, W1, degp)

    accp1 = scatter(hs1, src_p, dst_p, zeros2)

    hs2 = mm(
        _tc_mid,
        in_specs=[
            pl.BlockSpec((NC, br, h), lambda i: (0, i, 0)),
            pl.BlockSpec((br, h), lambda i: (i, 0)),
            pl.BlockSpec((NC, br), lambda i: (0, i)),
            pl.BlockSpec((1, h), lambda i: (0, 0)),
            pl.BlockSpec((h, h), lambda i: (0, 0)),
        ],
        out_specs=pl.BlockSpec((br, h), lambda i: (i, 0)),
        out_shape=jax.ShapeDtypeStruct((n, h), fdt),
    )(accp1, hs1, degp, b1.reshape(1, h), W2)

    accp2 = scatter(hs2, src_p, dst_p, zeros2)

    out = mm(
        _tc_final,
        in_specs=[
            pl.BlockSpec((NC, br, h), lambda i: (0, i, 0)),
            pl.BlockSpec((br, h), lambda i: (i, 0)),
            pl.BlockSpec((NC, br), lambda i: (0, i)),
            pl.BlockSpec((1, h), lambda i: (0, 0)),
            pl.BlockSpec((h, c), lambda i: (0, 0)),
            pl.BlockSpec((1, c), lambda i: (0, 0)),
        ],
        out_specs=pl.BlockSpec((br, c), lambda i: (i, 0)),
        out_shape=jax.ShapeDtypeStruct((n, c), jnp.float32),
    )(accp2, hs2, degp, b2.reshape(1, h), Wlin, blin.reshape(1, c))

    return out


# speedup vs baseline: 1.2469x; 1.2469x over previous
import functools

import jax
import jax.numpy as jnp
from jax import lax
from jax.experimental import pallas as pl
from jax.experimental.pallas import tpu as pltpu
from jax.experimental.pallas import tpu_sc as plsc

NC = 2
NS = 16
NW = NC * NS
CHUNK = 128


def _sc_mesh():
    return plsc.VectorSubcoreMesh(
        core_axis_name="c", subcore_axis_name="s", num_cores=NC, num_subcores=NS
    )


def _make_degree_call(acc_n, ch):
    rps = acc_n // NS

    def body(dst_hbm, zeros_hbm, out_hbm, idx_d, ones_v, degsh, sem):
        cid = lax.axis_index("c")
        sid = lax.axis_index("s")
        wid = sid * NC + cid
        pltpu.sync_copy(
            zeros_hbm.at[pl.ds(sid * rps, rps)], degsh.at[pl.ds(sid * rps, rps)]
        )
        pltpu.sync_copy(dst_hbm.at[wid], idx_d)
        for t in range(CHUNK // 16):
            ones_v[pl.ds(t * 16, 16)] = jnp.ones((16,), jnp.float32)
        plsc.subcore_barrier()

        def step(j, c):
            pltpu.sync_copy(ones_v, degsh.at[idx_d.at[j]], add=True)
            return c

        lax.fori_loop(0, ch, step, 0)
        plsc.subcore_barrier()
        pltpu.sync_copy(
            degsh.at[pl.ds(sid * rps, rps)],
            out_hbm.at[cid].at[pl.ds(sid * rps, rps)],
        )

    return pl.kernel(
        body,
        out_type=jax.ShapeDtypeStruct((NC, acc_n), jnp.float32),
        mesh=_sc_mesh(),
        scratch_types=[
            pltpu.VMEM((ch, CHUNK), jnp.int32),
            pltpu.VMEM((CHUNK,), jnp.float32),
            pltpu.VMEM_SHARED((acc_n,), jnp.float32),
            pltpu.SemaphoreType.DMA,
        ],
    )


def _make_scatter_call(acc_n, ch, h):
    rps = acc_n // NS

    assert ch % 4 == 0
    sch = ch // 2

    def body(feat_hbm, src_hbm, dst_hbm, zeros_hbm, out_hbm,
             idx_s, idx_d, rows0, rows1, accsh, gsem0, gsem1, ssem0, ssem1):
        cid = lax.axis_index("c")
        sid = lax.axis_index("s")
        wid = sid * NC + cid
        @pl.when(cid == 0)
        def _():
            pltpu.sync_copy(
                feat_hbm.at[pl.ds(sid * rps, rps)], accsh.at[pl.ds(sid * rps, rps)]
            )

        @pl.when(cid != 0)
        def _():
            pltpu.sync_copy(
                zeros_hbm.at[pl.ds(sid * rps, rps)], accsh.at[pl.ds(sid * rps, rps)]
            )

        plsc.subcore_barrier()

        for phase in range(2):
            base = phase * sch
            pltpu.sync_copy(src_hbm.at[wid].at[pl.ds(base, sch)], idx_s)
            pltpu.sync_copy(dst_hbm.at[wid].at[pl.ds(base, sch)], idx_d)
            pltpu.async_copy(feat_hbm.at[idx_s.at[0]], rows0, gsem0)
            pltpu.async_copy(feat_hbm.at[idx_s.at[1]], rows1, gsem1)

            def step(i, c):
                j0 = 2 * i
                j1 = 2 * i + 1
                pltpu.make_async_copy(
                    feat_hbm.at[idx_s.at[j0]], rows0, gsem0
                ).wait()
                pltpu.sync_copy(rows0, accsh.at[idx_d.at[j0]], add=True)

                @pl.when(j0 + 2 < sch)
                def _():
                    pltpu.async_copy(feat_hbm.at[idx_s.at[j0 + 2]], rows0, gsem0)

                pltpu.make_async_copy(
                    feat_hbm.at[idx_s.at[j1]], rows1, gsem1
                ).wait()
                pltpu.sync_copy(rows1, accsh.at[idx_d.at[j1]], add=True)

                @pl.when(j1 + 2 < sch)
                def _():
                    pltpu.async_copy(feat_hbm.at[idx_s.at[j1 + 2]], rows1, gsem1)

                return c

            lax.fori_loop(0, sch // 2, step, 0)
        plsc.subcore_barrier()
        pltpu.sync_copy(
            accsh.at[pl.ds(sid * rps, rps)],
            out_hbm.at[cid].at[pl.ds(sid * rps, rps)],
        )

    return pl.kernel(
        body,
        out_type=jax.ShapeDtypeStruct((NC, acc_n, h), jnp.float32),
        mesh=_sc_mesh(),
        scratch_types=[
            pltpu.VMEM((sch, CHUNK), jnp.int32),
            pltpu.VMEM((sch, CHUNK), jnp.int32),
            pltpu.VMEM((CHUNK, h), jnp.float32),
            pltpu.VMEM((CHUNK, h), jnp.float32),
            pltpu.VMEM_SHARED((acc_n, h), jnp.float32),
            pltpu.SemaphoreType.DMA,
            pltpu.SemaphoreType.DMA,
            pltpu.SemaphoreType.DMA,
            pltpu.SemaphoreType.DMA,
        ],
    )


def _tc_pre(x_ref, w_ref, degp_ref, out_ref):
    dis = lax.rsqrt(degp_ref[0, :] + degp_ref[1, :] + 1.0)
    hval = jnp.dot(x_ref[...], w_ref[...], preferred_element_type=jnp.float32)
    out_ref[...] = (hval * dis[:, None]).astype(out_ref.dtype)


def _tc_mid(accp_ref, degp_ref, b_ref, w_ref, out_ref):
    dis = lax.rsqrt(degp_ref[0, :] + degp_ref[1, :] + 1.0)
    acc = accp_ref[0] + accp_ref[1]
    x1 = jnp.maximum(acc * dis[:, None] + b_ref[...], 0.0)
    h2 = jnp.dot(x1, w_ref[...], preferred_element_type=jnp.float32)
    out_ref[...] = (h2 * dis[:, None]).astype(out_ref.dtype)


def _tc_final(accp_ref, degp_ref, b_ref, w_ref, blin_ref, out_ref):
    dis = lax.rsqrt(degp_ref[0, :] + degp_ref[1, :] + 1.0)
    acc = accp_ref[0] + accp_ref[1]
    x2 = jnp.maximum(acc * dis[:, None] + b_ref[...], 0.0)
    logits = jnp.dot(x2, w_ref[...], preferred_element_type=jnp.float32)
    logits = logits + blin_ref[...]
    m = jnp.max(logits, axis=1, keepdims=True)
    s = logits - m
    lse = jnp.log(jnp.sum(jnp.exp(s), axis=1, keepdims=True))
    out_ref[...] = s - lse


def kernel(x, edge_index, W1, b1, W2, b2, Wlin, blin):
    n, f = x.shape
    h = W1.shape[1]
    c = Wlin.shape[1]
    e = edge_index.shape[1]

    acc_n = -(-(n + 1) // (NS * CHUNK)) * (NS * CHUNK)
    dr = acc_n - n

    idt = edge_index.dtype
    if e % NW == 0:
        epw = e // NW
        ch = -(-epw // CHUNK)
        ch = -(-ch // 4) * 4
        pad = ch * CHUNK - epw
        w_ids = jnp.arange(NW, dtype=idt)[:, None]
        k_ids = jnp.arange(pad, dtype=idt)[None, :]
        pad_dst = (n + (k_ids + w_ids * 8) % dr).astype(idt)
        pad_src = ((k_ids * 61 + w_ids * 7) % n).astype(idt)
        src_p = jnp.concatenate(
            [edge_index[0].reshape(NW, epw), pad_src], axis=1
        ).reshape(NW, ch, CHUNK)
        dst_p = jnp.concatenate(
            [edge_index[1].reshape(NW, epw), pad_dst], axis=1
        ).reshape(NW, ch, CHUNK)
    else:
        ch = -(-e // (NW * CHUNK))
        ch = -(-ch // 4) * 4
        pad = NW * ch * CHUNK - e
        k_ids = jnp.arange(pad, dtype=idt)
        src_p = jnp.concatenate(
            [edge_index[0], (k_ids * 61) % n]
        ).reshape(NW, ch, CHUNK)
        dst_p = jnp.concatenate(
            [edge_index[1], n + k_ids % dr]
        ).reshape(NW, ch, CHUNK)
    fdt = jnp.float32
    zeros2 = jnp.zeros((acc_n, h), fdt)
    zeros1 = jnp.zeros((acc_n,), jnp.float32)

    degp = _make_degree_call(acc_n, ch)(dst_p, zeros1)
    scatter = _make_scatter_call(acc_n, ch, h)

    br = 1280
    grid = (acc_n // br,)
    mm = functools.partial(
        pl.pallas_call,
        grid=grid,
    )

    hs1 = mm(
        _tc_pre,
        in_specs=[
            pl.BlockSpec((br, f), lambda i: (i, 0)),
            pl.BlockSpec((f, h), lambda i: (0, 0)),
            pl.BlockSpec((NC, br), lambda i: (0, i)),
        ],
        out_specs=pl.BlockSpec((br, h), lambda i: (i, 0)),
        out_shape=jax.ShapeDtypeStruct((acc_n, h), fdt),
    )(x, W1, degp)

    accp1 = scatter(hs1, src_p, dst_p, zeros2)

    hs2 = mm(
        _tc_mid,
        in_specs=[
            pl.BlockSpec((NC, br, h), lambda i: (0, i, 0)),
            pl.BlockSpec((NC, br), lambda i: (0, i)),
            pl.BlockSpec((1, h), lambda i: (0, 0)),
            pl.BlockSpec((h, h), lambda i: (0, 0)),
        ],
        out_specs=pl.BlockSpec((br, h), lambda i: (i, 0)),
        out_shape=jax.ShapeDtypeStruct((acc_n, h), fdt),
    )(accp1, degp, b1.reshape(1, h), W2)

    accp2 = scatter(hs2, src_p, dst_p, zeros2)

    out = mm(
        _tc_final,
        in_specs=[
            pl.BlockSpec((NC, br, h), lambda i: (0, i, 0)),
            pl.BlockSpec((NC, br), lambda i: (0, i)),
            pl.BlockSpec((1, h), lambda i: (0, 0)),
            pl.BlockSpec((h, c), lambda i: (0, 0)),
            pl.BlockSpec((1, c), lambda i: (0, 0)),
        ],
        out_specs=pl.BlockSpec((br, c), lambda i: (i, 0)),
        out_shape=jax.ShapeDtypeStruct((n, c), jnp.float32),
    )(accp2, degp, b2.reshape(1, h), Wlin, blin.reshape(1, c))

    return out


# speedup vs baseline: 1.3039x; 1.0458x over previous
import functools

import jax
import jax.numpy as jnp
from jax import lax
from jax.experimental import pallas as pl
from jax.experimental.pallas import tpu as pltpu
from jax.experimental.pallas import tpu_sc as plsc

NC = 2
NS = 16
NW = NC * NS
CHUNK = 128


def _sc_mesh():
    return plsc.VectorSubcoreMesh(
        core_axis_name="c", subcore_axis_name="s", num_cores=NC, num_subcores=NS
    )


def _make_degree_call(acc_n, ch):
    rps = acc_n // NS

    def body(ep_hbm, out_hbm, idx_d, ones_v, zb, degsh, sem):
        cid = lax.axis_index("c")
        sid = lax.axis_index("s")
        wid = sid * NC + cid
        for t in range(CHUNK // 16):
            zb[pl.ds(t * 16, 16)] = jnp.zeros((16,), jnp.float32)

        def zstep(i, c):
            pltpu.sync_copy(zb, degsh.at[pl.ds(sid * rps + i * CHUNK, CHUNK)])
            return c

        lax.fori_loop(0, rps // CHUNK, zstep, 0)
        pltpu.sync_copy(ep_hbm.at[1].at[wid], idx_d)
        for t in range(CHUNK // 16):
            ones_v[pl.ds(t * 16, 16)] = jnp.ones((16,), jnp.float32)
        plsc.subcore_barrier()

        def step(j, c):
            pltpu.sync_copy(ones_v, degsh.at[idx_d.at[j]], add=True)
            return c

        lax.fori_loop(0, ch, step, 0)
        plsc.subcore_barrier()
        pltpu.sync_copy(
            degsh.at[pl.ds(sid * rps, rps)],
            out_hbm.at[cid].at[pl.ds(sid * rps, rps)],
        )

    return pl.kernel(
        body,
        out_type=jax.ShapeDtypeStruct((NC, acc_n), jnp.float32),
        mesh=_sc_mesh(),
        scratch_types=[
            pltpu.VMEM((ch, CHUNK), jnp.int32),
            pltpu.VMEM((CHUNK,), jnp.float32),
            pltpu.VMEM((CHUNK,), jnp.float32),
            pltpu.VMEM_SHARED((acc_n,), jnp.float32),
            pltpu.SemaphoreType.DMA,
        ],
    )


def _make_scatter_call(acc_n, ch, h):
    rps = acc_n // NS

    assert ch % 4 == 0
    sch = ch // 2
    ZROWS = 32
    assert rps % ZROWS == 0

    def body(feat_hbm, ep_hbm, out_hbm,
             idx_s, idx_d, rows0, rows1, zb, accsh,
             gsem0, gsem1, ssem0, ssem1):
        cid = lax.axis_index("c")
        sid = lax.axis_index("s")
        wid = sid * NC + cid
        @pl.when(cid == 0)
        def _():
            pltpu.sync_copy(
                feat_hbm.at[pl.ds(sid * rps, rps)], accsh.at[pl.ds(sid * rps, rps)]
            )

        @pl.when(cid != 0)
        def _():
            for r in range(ZROWS):
                for t in range(h // 16):
                    zb[r, pl.ds(t * 16, 16)] = jnp.zeros((16,), jnp.float32)

            def zstep(i, c):
                pltpu.sync_copy(
                    zb, accsh.at[pl.ds(sid * rps + i * ZROWS, ZROWS)]
                )
                return c

            lax.fori_loop(0, rps // ZROWS, zstep, 0)

        plsc.subcore_barrier()

        for phase in range(2):
            base = phase * sch
            pltpu.sync_copy(ep_hbm.at[0].at[wid].at[pl.ds(base, sch)], idx_s)
            pltpu.sync_copy(ep_hbm.at[1].at[wid].at[pl.ds(base, sch)], idx_d)
            pltpu.async_copy(feat_hbm.at[idx_s.at[0]], rows0, gsem0)
            pltpu.async_copy(feat_hbm.at[idx_s.at[1]], rows1, gsem1)

            def step(i, c):
                j0 = 2 * i
                j1 = 2 * i + 1
                pltpu.make_async_copy(
                    feat_hbm.at[idx_s.at[j0]], rows0, gsem0
                ).wait()
                pltpu.sync_copy(rows0, accsh.at[idx_d.at[j0]], add=True)

                @pl.when(j0 + 2 < sch)
                def _():
                    pltpu.async_copy(feat_hbm.at[idx_s.at[j0 + 2]], rows0, gsem0)

                pltpu.make_async_copy(
                    feat_hbm.at[idx_s.at[j1]], rows1, gsem1
                ).wait()
                pltpu.sync_copy(rows1, accsh.at[idx_d.at[j1]], add=True)

                @pl.when(j1 + 2 < sch)
                def _():
                    pltpu.async_copy(feat_hbm.at[idx_s.at[j1 + 2]], rows1, gsem1)

                return c

            lax.fori_loop(0, sch // 2, step, 0)
        plsc.subcore_barrier()
        pltpu.sync_copy(
            accsh.at[pl.ds(sid * rps, rps)],
            out_hbm.at[cid].at[pl.ds(sid * rps, rps)],
        )

    return pl.kernel(
        body,
        out_type=jax.ShapeDtypeStruct((NC, acc_n, h), jnp.float32),
        mesh=_sc_mesh(),
        scratch_types=[
            pltpu.VMEM((sch, CHUNK), jnp.int32),
            pltpu.VMEM((sch, CHUNK), jnp.int32),
            pltpu.VMEM((CHUNK, h), jnp.float32),
            pltpu.VMEM((CHUNK, h), jnp.float32),
            pltpu.VMEM((32, h), jnp.float32),
            pltpu.VMEM_SHARED((acc_n, h), jnp.float32),
            pltpu.SemaphoreType.DMA,
            pltpu.SemaphoreType.DMA,
            pltpu.SemaphoreType.DMA,
            pltpu.SemaphoreType.DMA,
        ],
    )


def _tc_pre(x_ref, w_ref, degp_ref, out_ref):
    dis = lax.rsqrt(degp_ref[0, :] + degp_ref[1, :] + 1.0)
    hval = jnp.dot(x_ref[...], w_ref[...], preferred_element_type=jnp.float32)
    out_ref[...] = (hval * dis[:, None]).astype(out_ref.dtype)


def _tc_mid(accp_ref, degp_ref, b_ref, w_ref, out_ref):
    dis = lax.rsqrt(degp_ref[0, :] + degp_ref[1, :] + 1.0)
    acc = accp_ref[0] + accp_ref[1]
    x1 = jnp.maximum(acc * dis[:, None] + b_ref[...], 0.0)
    h2 = jnp.dot(x1, w_ref[...], preferred_element_type=jnp.float32)
    out_ref[...] = (h2 * dis[:, None]).astype(out_ref.dtype)


def _tc_final(accp_ref, degp_ref, b_ref, w_ref, blin_ref, out_ref):
    dis = lax.rsqrt(degp_ref[0, :] + degp_ref[1, :] + 1.0)
    acc = accp_ref[0] + accp_ref[1]
    x2 = jnp.maximum(acc * dis[:, None] + b_ref[...], 0.0)
    logits = jnp.dot(x2, w_ref[...], preferred_element_type=jnp.float32)
    logits = logits + blin_ref[...]
    m = jnp.max(logits, axis=1, keepdims=True)
    s = logits - m
    lse = jnp.log(jnp.sum(jnp.exp(s), axis=1, keepdims=True))
    out_ref[...] = s - lse


def kernel(x, edge_index, W1, b1, W2, b2, Wlin, blin):
    n, f = x.shape
    h = W1.shape[1]
    c = Wlin.shape[1]
    e = edge_index.shape[1]

    acc_n = -(-(n + 1) // (NS * CHUNK)) * (NS * CHUNK)
    dr = acc_n - n

    idt = edge_index.dtype
    if e % NW == 0:
        epw = e // NW
        ch = -(-epw // CHUNK)
        ch = -(-ch // 4) * 4
        pad = ch * CHUNK - epw
        w_ids = jnp.arange(NW, dtype=idt)[None, :, None]
        k_ids = jnp.arange(pad, dtype=idt)[None, None, :]
        pad_src = ((k_ids * 61 + w_ids * 7) % n).astype(idt)
        pad_dst = (n + (k_ids + w_ids * 8) % dr).astype(idt)
        pads = jnp.concatenate([pad_src, pad_dst], axis=0)
        ep = jnp.concatenate(
            [edge_index.reshape(2, NW, epw), pads], axis=2
        ).reshape(2, NW, ch, CHUNK)
    else:
        ch = -(-e // (NW * CHUNK))
        ch = -(-ch // 4) * 4
        pad = NW * ch * CHUNK - e
        k_ids = jnp.arange(pad, dtype=idt)[None, :]
        pad_src = ((k_ids * 61) % n).astype(idt)
        pad_dst = (n + k_ids % dr).astype(idt)
        pads = jnp.concatenate([pad_src, pad_dst], axis=0)
        ep = jnp.concatenate([edge_index, pads], axis=1).reshape(
            2, NW, ch, CHUNK
        )

    fdt = jnp.float32
    degp = _make_degree_call(acc_n, ch)(ep)
    scatter = _make_scatter_call(acc_n, ch, h)

    br = 1280
    grid = (acc_n // br,)
    mm = functools.partial(
        pl.pallas_call,
        grid=grid,
    )

    hs1 = mm(
        _tc_pre,
        in_specs=[
            pl.BlockSpec((br, f), lambda i: (i, 0)),
            pl.BlockSpec((f, h), lambda i: (0, 0)),
            pl.BlockSpec((NC, br), lambda i: (0, i)),
        ],
        out_specs=pl.BlockSpec((br, h), lambda i: (i, 0)),
        out_shape=jax.ShapeDtypeStruct((acc_n, h), fdt),
    )(x, W1, degp)

    accp1 = scatter(hs1, ep)

    hs2 = mm(
        _tc_mid,
        in_specs=[
            pl.BlockSpec((NC, br, h), lambda i: (0, i, 0)),
            pl.BlockSpec((NC, br), lambda i: (0, i)),
            pl.BlockSpec((1, h), lambda i: (0, 0)),
            pl.BlockSpec((h, h), lambda i: (0, 0)),
        ],
        out_specs=pl.BlockSpec((br, h), lambda i: (i, 0)),
        out_shape=jax.ShapeDtypeStruct((acc_n, h), fdt),
    )(accp1, degp, b1.reshape(1, h), W2)

    accp2 = scatter(hs2, ep)

    out = mm(
        _tc_final,
        in_specs=[
            pl.BlockSpec((NC, br, h), lambda i: (0, i, 0)),
            pl.BlockSpec((NC, br), lambda i: (0, i)),
            pl.BlockSpec((1, h), lambda i: (0, 0)),
            pl.BlockSpec((h, c), lambda i: (0, 0)),
            pl.BlockSpec((1, c), lambda i: (0, 0)),
        ],
        out_specs=pl.BlockSpec((br, c), lambda i: (i, 0)),
        out_shape=jax.ShapeDtypeStruct((n, c), jnp.float32),
    )(accp2, degp, b2.reshape(1, h), Wlin, blin.reshape(1, c))

    return out


# speedup vs baseline: 1.3512x; 1.0362x over previous
import functools

import jax
import jax.numpy as jnp
from jax import lax
from jax.experimental import pallas as pl
from jax.experimental.pallas import tpu as pltpu
from jax.experimental.pallas import tpu_sc as plsc

NC = 2
NS = 16
NW = NC * NS
CHUNK = 128


def _sc_mesh():
    return plsc.VectorSubcoreMesh(
        core_axis_name="c", subcore_axis_name="s", num_cores=NC, num_subcores=NS
    )


def _make_degree_call(acc_n, ch):
    rps = acc_n // NS

    def body(ep_hbm, out_hbm, idx_d, ones_v, zb, degsh, sem):
        cid = lax.axis_index("c")
        sid = lax.axis_index("s")
        wid = sid * NC + cid
        for t in range(CHUNK // 16):
            zb[pl.ds(t * 16, 16)] = jnp.zeros((16,), jnp.float32)

        def zstep(i, c):
            pltpu.sync_copy(zb, degsh.at[pl.ds(sid * rps + i * CHUNK, CHUNK)])
            return c

        lax.fori_loop(0, rps // CHUNK, zstep, 0)
        pltpu.sync_copy(ep_hbm.at[1].at[wid], idx_d)
        for t in range(CHUNK // 16):
            ones_v[pl.ds(t * 16, 16)] = jnp.ones((16,), jnp.float32)
        plsc.subcore_barrier()

        def step(j, c):
            pltpu.async_copy(ones_v, degsh.at[idx_d.at[j]], sem, add=True)
            return c

        lax.fori_loop(0, ch, step, 0)

        def dstep(j, c):
            pltpu.make_async_copy(ones_v, degsh.at[idx_d.at[0]], sem).wait()
            return c

        lax.fori_loop(0, ch, dstep, 0)
        plsc.subcore_barrier()
        pltpu.sync_copy(
            degsh.at[pl.ds(sid * rps, rps)],
            out_hbm.at[cid].at[pl.ds(sid * rps, rps)],
        )

    return pl.kernel(
        body,
        out_type=jax.ShapeDtypeStruct((NC, acc_n), jnp.float32),
        mesh=_sc_mesh(),
        scratch_types=[
            pltpu.VMEM((ch, CHUNK), jnp.int32),
            pltpu.VMEM((CHUNK,), jnp.float32),
            pltpu.VMEM((CHUNK,), jnp.float32),
            pltpu.VMEM_SHARED((acc_n,), jnp.float32),
            pltpu.SemaphoreType.DMA,
        ],
    )


def _make_scatter_call(acc_n, ch, h):
    rps = acc_n // NS

    assert ch % 4 == 0
    sch = ch // 2
    ZROWS = 32
    assert rps % ZROWS == 0

    def body(feat_hbm, ep_hbm, out_hbm,
             idx_s, idx_d, rows0, rows1, zb, accsh,
             gsem0, gsem1, ssem0, ssem1):
        cid = lax.axis_index("c")
        sid = lax.axis_index("s")
        wid = sid * NC + cid
        @pl.when(cid == 0)
        def _():
            pltpu.sync_copy(
                feat_hbm.at[pl.ds(sid * rps, rps)], accsh.at[pl.ds(sid * rps, rps)]
            )

        @pl.when(cid != 0)
        def _():
            for r in range(ZROWS):
                for t in range(h // 16):
                    zb[r, pl.ds(t * 16, 16)] = jnp.zeros((16,), jnp.float32)

            def zstep(i, c):
                pltpu.sync_copy(
                    zb, accsh.at[pl.ds(sid * rps + i * ZROWS, ZROWS)]
                )
                return c

            lax.fori_loop(0, rps // ZROWS, zstep, 0)

        plsc.subcore_barrier()

        for phase in range(2):
            base = phase * sch
            pltpu.sync_copy(ep_hbm.at[0].at[wid].at[pl.ds(base, sch)], idx_s)
            pltpu.sync_copy(ep_hbm.at[1].at[wid].at[pl.ds(base, sch)], idx_d)
            pltpu.async_copy(feat_hbm.at[idx_s.at[0]], rows0, gsem0)
            pltpu.async_copy(feat_hbm.at[idx_s.at[1]], rows1, gsem1)

            def step(i, c):
                j0 = 2 * i
                j1 = 2 * i + 1
                pltpu.make_async_copy(
                    feat_hbm.at[idx_s.at[j0]], rows0, gsem0
                ).wait()
                pltpu.sync_copy(rows0, accsh.at[idx_d.at[j0]], add=True)

                @pl.when(j0 + 2 < sch)
                def _():
                    pltpu.async_copy(feat_hbm.at[idx_s.at[j0 + 2]], rows0, gsem0)

                pltpu.make_async_copy(
                    feat_hbm.at[idx_s.at[j1]], rows1, gsem1
                ).wait()
                pltpu.sync_copy(rows1, accsh.at[idx_d.at[j1]], add=True)

                @pl.when(j1 + 2 < sch)
                def _():
                    pltpu.async_copy(feat_hbm.at[idx_s.at[j1 + 2]], rows1, gsem1)

                return c

            lax.fori_loop(0, sch // 2, step, 0)
        plsc.subcore_barrier()
        pltpu.sync_copy(
            accsh.at[pl.ds(sid * rps, rps)],
            out_hbm.at[cid].at[pl.ds(sid * rps, rps)],
        )

    return pl.kernel(
        body,
        out_type=jax.ShapeDtypeStruct((NC, acc_n, h), jnp.float32),
        mesh=_sc_mesh(),
        scratch_types=[
            pltpu.VMEM((sch, CHUNK), jnp.int32),
            pltpu.VMEM((sch, CHUNK), jnp.int32),
            pltpu.VMEM((CHUNK, h), jnp.float32),
            pltpu.VMEM((CHUNK, h), jnp.float32),
            pltpu.VMEM((32, h), jnp.float32),
            pltpu.VMEM_SHARED((acc_n, h), jnp.float32),
            pltpu.SemaphoreType.DMA,
            pltpu.SemaphoreType.DMA,
            pltpu.SemaphoreType.DMA,
            pltpu.SemaphoreType.DMA,
        ],
    )


def _tc_pre(x_ref, w_ref, degp_ref, out_ref):
    dis = lax.rsqrt(degp_ref[0, :] + degp_ref[1, :] + 1.0)
    hval = jnp.dot(x_ref[...], w_ref[...], preferred_element_type=jnp.float32)
    out_ref[...] = (hval * dis[:, None]).astype(out_ref.dtype)


def _tc_mid(accp_ref, degp_ref, b_ref, w_ref, out_ref):
    dis = lax.rsqrt(degp_ref[0, :] + degp_ref[1, :] + 1.0)
    acc = accp_ref[0] + accp_ref[1]
    x1 = jnp.maximum(acc * dis[:, None] + b_ref[...], 0.0)
    h2 = jnp.dot(x1, w_ref[...], preferred_element_type=jnp.float32)
    out_ref[...] = (h2 * dis[:, None]).astype(out_ref.dtype)


def _tc_final(accp_ref, degp_ref, b_ref, w_ref, blin_ref, out_ref):
    dis = lax.rsqrt(degp_ref[0, :] + degp_ref[1, :] + 1.0)
    acc = accp_ref[0] + accp_ref[1]
    x2 = jnp.maximum(acc * dis[:, None] + b_ref[...], 0.0)
    logits = jnp.dot(x2, w_ref[...], preferred_element_type=jnp.float32)
    logits = logits + blin_ref[...]
    m = jnp.max(logits, axis=1, keepdims=True)
    s = logits - m
    lse = jnp.log(jnp.sum(jnp.exp(s), axis=1, keepdims=True))
    out_ref[...] = (s - lse).T


def kernel(x, edge_index, W1, b1, W2, b2, Wlin, blin):
    n, f = x.shape
    h = W1.shape[1]
    c = Wlin.shape[1]
    e = edge_index.shape[1]

    acc_n = -(-(n + 1) // (NS * CHUNK)) * (NS * CHUNK)
    dr = acc_n - n

    idt = edge_index.dtype
    if e % NW == 0:
        epw = e // NW
        ch = -(-epw // CHUNK)
        ch = -(-ch // 4) * 4
        pad = ch * CHUNK - epw
        w_ids = jnp.arange(NW, dtype=idt)[None, :, None]
        k_ids = jnp.arange(pad, dtype=idt)[None, None, :]
        pad_src = ((k_ids * 61 + w_ids * 7) % n).astype(idt)
        pad_dst = (n + (k_ids + w_ids * 8) % dr).astype(idt)
        pads = jnp.concatenate([pad_src, pad_dst], axis=0)
        ep = jnp.concatenate(
            [edge_index.reshape(2, NW, epw), pads], axis=2
        ).reshape(2, NW, ch, CHUNK)
    else:
        ch = -(-e // (NW * CHUNK))
        ch = -(-ch // 4) * 4
        pad = NW * ch * CHUNK - e
        k_ids = jnp.arange(pad, dtype=idt)[None, :]
        pad_src = ((k_ids * 61) % n).astype(idt)
        pad_dst = (n + k_ids % dr).astype(idt)
        pads = jnp.concatenate([pad_src, pad_dst], axis=0)
        ep = jnp.concatenate([edge_index, pads], axis=1).reshape(
            2, NW, ch, CHUNK
        )

    fdt = jnp.float32
    degp = _make_degree_call(acc_n, ch)(ep)
    scatter = _make_scatter_call(acc_n, ch, h)

    br = 1280
    grid = (acc_n // br,)
    mm = functools.partial(
        pl.pallas_call,
        grid=grid,
    )

    hs1 = mm(
        _tc_pre,
        in_specs=[
            pl.BlockSpec((br, f), lambda i: (i, 0)),
            pl.BlockSpec((f, h), lambda i: (0, 0)),
            pl.BlockSpec((NC, br), lambda i: (0, i)),
        ],
        out_specs=pl.BlockSpec((br, h), lambda i: (i, 0)),
        out_shape=jax.ShapeDtypeStruct((acc_n, h), fdt),
    )(x, W1, degp)

    accp1 = scatter(hs1, ep)

    hs2 = mm(
        _tc_mid,
        in_specs=[
            pl.BlockSpec((NC, br, h), lambda i: (0, i, 0)),
            pl.BlockSpec((NC, br), lambda i: (0, i)),
            pl.BlockSpec((1, h), lambda i: (0, 0)),
            pl.BlockSpec((h, h), lambda i: (0, 0)),
        ],
        out_specs=pl.BlockSpec((br, h), lambda i: (i, 0)),
        out_shape=jax.ShapeDtypeStruct((acc_n, h), fdt),
    )(accp1, degp, b1.reshape(1, h), W2)

    accp2 = scatter(hs2, ep)

    out = mm(
        _tc_final,
        in_specs=[
            pl.BlockSpec((NC, br, h), lambda i: (0, i, 0)),
            pl.BlockSpec((NC, br), lambda i: (0, i)),
            pl.BlockSpec((1, h), lambda i: (0, 0)),
            pl.BlockSpec((h, c), lambda i: (0, 0)),
            pl.BlockSpec((1, c), lambda i: (0, 0)),
        ],
        out_specs=pl.BlockSpec((c, br), lambda i: (0, i)),
        out_shape=jax.ShapeDtypeStruct((c, n), jnp.float32),
    )(accp2, degp, b2.reshape(1, h), Wlin, blin.reshape(1, c))

    return out.T


# speedup vs baseline: 1.3939x; 1.0316x over previous
import functools

import jax
import jax.numpy as jnp
from jax import lax
from jax.experimental import pallas as pl
from jax.experimental.pallas import tpu as pltpu
from jax.experimental.pallas import tpu_sc as plsc

NC = 2
NS = 16
NW = NC * NS
CHUNK = 128


def _sc_mesh():
    return plsc.VectorSubcoreMesh(
        core_axis_name="c", subcore_axis_name="s", num_cores=NC, num_subcores=NS
    )


def _make_degree_call(acc_n, ch):
    rps = acc_n // NS

    def body(ep_hbm, out_hbm, idx_d, ones_v, zb, degsh, sem):
        cid = lax.axis_index("c")
        sid = lax.axis_index("s")
        wid = sid * NC + cid
        for t in range(CHUNK // 16):
            zb[pl.ds(t * 16, 16)] = jnp.zeros((16,), jnp.float32)

        def zstep(i, c):
            pltpu.sync_copy(zb, degsh.at[pl.ds(sid * rps + i * CHUNK, CHUNK)])
            return c

        lax.fori_loop(0, rps // CHUNK, zstep, 0)
        pltpu.sync_copy(ep_hbm.at[1].at[wid], idx_d)
        for t in range(CHUNK // 16):
            ones_v[pl.ds(t * 16, 16)] = jnp.ones((16,), jnp.float32)
        plsc.subcore_barrier()

        def step(j, c):
            pltpu.async_copy(ones_v, degsh.at[idx_d.at[j]], sem, add=True)
            return c

        lax.fori_loop(0, ch, step, 0)

        def dstep(j, c):
            pltpu.make_async_copy(ones_v, degsh.at[idx_d.at[0]], sem).wait()
            return c

        lax.fori_loop(0, ch, dstep, 0)
        plsc.subcore_barrier()
        pltpu.sync_copy(
            degsh.at[pl.ds(sid * rps, rps)],
            out_hbm.at[cid].at[pl.ds(sid * rps, rps)],
        )

    return pl.kernel(
        body,
        out_type=jax.ShapeDtypeStruct((NC, acc_n), jnp.float32),
        mesh=_sc_mesh(),
        scratch_types=[
            pltpu.VMEM((ch, CHUNK), jnp.int32),
            pltpu.VMEM((CHUNK,), jnp.float32),
            pltpu.VMEM((CHUNK,), jnp.float32),
            pltpu.VMEM_SHARED((acc_n,), jnp.float32),
            pltpu.SemaphoreType.DMA,
        ],
    )


def _make_scatter_call(acc_n, ch, h):
    rps = acc_n // NS

    assert ch % 4 == 0
    sch = ch // 2
    ZROWS = 32
    assert rps % ZROWS == 0

    def body(feat_hbm, ep_hbm, out_hbm,
             idx_s, idx_d, rows0, rows1, zb, accsh,
             gsem0, gsem1, ssem0, ssem1):
        cid = lax.axis_index("c")
        sid = lax.axis_index("s")
        wid = sid * NC + cid
        @pl.when(cid == 0)
        def _():
            pltpu.async_copy(
                feat_hbm.at[pl.ds(sid * rps, rps)],
                accsh.at[pl.ds(sid * rps, rps)],
                ssem0,
            )

        @pl.when(cid != 0)
        def _():
            for r in range(ZROWS):
                for t in range(h // 16):
                    zb[r, pl.ds(t * 16, 16)] = jnp.zeros((16,), jnp.float32)

            def zstep(i, c):
                pltpu.async_copy(
                    zb, accsh.at[pl.ds(sid * rps + i * ZROWS, ZROWS)], ssem0
                )
                return c

            lax.fori_loop(0, rps // ZROWS, zstep, 0)

        pltpu.sync_copy(ep_hbm.at[0].at[wid].at[pl.ds(0, sch)], idx_s)
        pltpu.sync_copy(ep_hbm.at[1].at[wid].at[pl.ds(0, sch)], idx_d)
        pltpu.async_copy(feat_hbm.at[idx_s.at[0]], rows0, gsem0)
        pltpu.async_copy(feat_hbm.at[idx_s.at[1]], rows1, gsem1)

        @pl.when(cid == 0)
        def _():
            pltpu.make_async_copy(
                feat_hbm.at[pl.ds(sid * rps, rps)],
                accsh.at[pl.ds(sid * rps, rps)],
                ssem0,
            ).wait()

        @pl.when(cid != 0)
        def _():
            def zdrain(i, c):
                pltpu.make_async_copy(
                    zb, accsh.at[pl.ds(sid * rps, ZROWS)], ssem0
                ).wait()
                return c

            lax.fori_loop(0, rps // ZROWS, zdrain, 0)

        plsc.subcore_barrier()

        for phase in range(2):
            base = phase * sch
            if phase > 0:
                pltpu.sync_copy(
                    ep_hbm.at[0].at[wid].at[pl.ds(base, sch)], idx_s
                )
                pltpu.sync_copy(
                    ep_hbm.at[1].at[wid].at[pl.ds(base, sch)], idx_d
                )
                pltpu.async_copy(feat_hbm.at[idx_s.at[0]], rows0, gsem0)
                pltpu.async_copy(feat_hbm.at[idx_s.at[1]], rows1, gsem1)

            def step(i, c):
                j0 = 2 * i
                j1 = 2 * i + 1
                pltpu.make_async_copy(
                    feat_hbm.at[idx_s.at[j0]], rows0, gsem0
                ).wait()
                pltpu.sync_copy(rows0, accsh.at[idx_d.at[j0]], add=True)

                @pl.when(j0 + 2 < sch)
                def _():
                    pltpu.async_copy(feat_hbm.at[idx_s.at[j0 + 2]], rows0, gsem0)

                pltpu.make_async_copy(
                    feat_hbm.at[idx_s.at[j1]], rows1, gsem1
                ).wait()
                pltpu.sync_copy(rows1, accsh.at[idx_d.at[j1]], add=True)

                @pl.when(j1 + 2 < sch)
                def _():
                    pltpu.async_copy(feat_hbm.at[idx_s.at[j1 + 2]], rows1, gsem1)

                return c

            lax.fori_loop(0, sch // 2, step, 0)
        plsc.subcore_barrier()
        pltpu.sync_copy(
            accsh.at[pl.ds(sid * rps, rps)],
            out_hbm.at[cid].at[pl.ds(sid * rps, rps)],
        )

    return pl.kernel(
        body,
        out_type=jax.ShapeDtypeStruct((NC, acc_n, h), jnp.float32),
        mesh=_sc_mesh(),
        scratch_types=[
            pltpu.VMEM((sch, CHUNK), jnp.int32),
            pltpu.VMEM((sch, CHUNK), jnp.int32),
            pltpu.VMEM((CHUNK, h), jnp.float32),
            pltpu.VMEM((CHUNK, h), jnp.float32),
            pltpu.VMEM((32, h), jnp.float32),
            pltpu.VMEM_SHARED((acc_n, h), jnp.float32),
            pltpu.SemaphoreType.DMA,
            pltpu.SemaphoreType.DMA,
            pltpu.SemaphoreType.DMA,
            pltpu.SemaphoreType.DMA,
        ],
    )


def _tc_pre(x_ref, w_ref, degp_ref, out_ref):
    dis = lax.rsqrt(degp_ref[0, :] + degp_ref[1, :] + 1.0)
    hval = jnp.dot(x_ref[...], w_ref[...], preferred_element_type=jnp.float32)
    out_ref[...] = (hval * dis[:, None]).astype(out_ref.dtype)


def _tc_mid(accp_ref, degp_ref, b_ref, w_ref, out_ref):
    dis = lax.rsqrt(degp_ref[0, :] + degp_ref[1, :] + 1.0)
    acc = accp_ref[0] + accp_ref[1]
    x1 = jnp.maximum(acc * dis[:, None] + b_ref[...], 0.0)
    h2 = jnp.dot(x1, w_ref[...], preferred_element_type=jnp.float32)
    out_ref[...] = (h2 * dis[:, None]).astype(out_ref.dtype)


def _tc_final(accp_ref, degp_ref, b_ref, w_ref, blin_ref, out_ref):
    dis = lax.rsqrt(degp_ref[0, :] + degp_ref[1, :] + 1.0)
    acc = accp_ref[0] + accp_ref[1]
    x2 = jnp.maximum(acc * dis[:, None] + b_ref[...], 0.0)
    logits = jnp.dot(x2, w_ref[...], preferred_element_type=jnp.float32)
    logits = logits + blin_ref[...]
    m = jnp.max(logits, axis=1, keepdims=True)
    s = logits - m
    lse = jnp.log(jnp.sum(jnp.exp(s), axis=1, keepdims=True))
    out_ref[...] = (s - lse).T


def kernel(x, edge_index, W1, b1, W2, b2, Wlin, blin):
    n, f = x.shape
    h = W1.shape[1]
    c = Wlin.shape[1]
    e = edge_index.shape[1]

    acc_n = -(-(n + 1) // (NS * CHUNK)) * (NS * CHUNK)
    dr = acc_n - n

    idt = edge_index.dtype
    if e % NW == 0:
        epw = e // NW
        ch = -(-epw // CHUNK)
        ch = -(-ch // 4) * 4
        pad = ch * CHUNK - epw
        w_ids = jnp.arange(NW, dtype=idt)[None, :, None]
        k_ids = jnp.arange(pad, dtype=idt)[None, None, :]
        pad_src = ((k_ids * 61 + w_ids * 7) % n).astype(idt)
        pad_dst = (n + (k_ids + w_ids * 8) % dr).astype(idt)
        pads = jnp.concatenate([pad_src, pad_dst], axis=0)
        ep = jnp.concatenate(
            [edge_index.reshape(2, NW, epw), pads], axis=2
        ).reshape(2, NW, ch, CHUNK)
    else:
        ch = -(-e // (NW * CHUNK))
        ch = -(-ch // 4) * 4
        pad = NW * ch * CHUNK - e
        k_ids = jnp.arange(pad, dtype=idt)[None, :]
        pad_src = ((k_ids * 61) % n).astype(idt)
        pad_dst = (n + k_ids % dr).astype(idt)
        pads = jnp.concatenate([pad_src, pad_dst], axis=0)
        ep = jnp.concatenate([edge_index, pads], axis=1).reshape(
            2, NW, ch, CHUNK
        )

    fdt = jnp.float32
    degp = _make_degree_call(acc_n, ch)(ep)
    scatter = _make_scatter_call(acc_n, ch, h)

    br = 1280
    grid = (acc_n // br,)
    mm = functools.partial(
        pl.pallas_call,
        grid=grid,
    )

    hs1 = mm(
        _tc_pre,
        in_specs=[
            pl.BlockSpec((br, f), lambda i: (i, 0)),
            pl.BlockSpec((f, h), lambda i: (0, 0)),
            pl.BlockSpec((NC, br), lambda i: (0, i)),
        ],
        out_specs=pl.BlockSpec((br, h), lambda i: (i, 0)),
        out_shape=jax.ShapeDtypeStruct((acc_n, h), fdt),
    )(x, W1, degp)

    accp1 = scatter(hs1, ep)

    hs2 = mm(
        _tc_mid,
        in_specs=[
            pl.BlockSpec((NC, br, h), lambda i: (0, i, 0)),
            pl.BlockSpec((NC, br), lambda i: (0, i)),
            pl.BlockSpec((1, h), lambda i: (0, 0)),
            pl.BlockSpec((h, h), lambda i: (0, 0)),
        ],
        out_specs=pl.BlockSpec((br, h), lambda i: (i, 0)),
        out_shape=jax.ShapeDtypeStruct((acc_n, h), fdt),
    )(accp1, degp, b1.reshape(1, h), W2)

    accp2 = scatter(hs2, ep)

    out = mm(
        _tc_final,
        in_specs=[
            pl.BlockSpec((NC, br, h), lambda i: (0, i, 0)),
            pl.BlockSpec((NC, br), lambda i: (0, i)),
            pl.BlockSpec((1, h), lambda i: (0, 0)),
            pl.BlockSpec((h, c), lambda i: (0, 0)),
            pl.BlockSpec((1, c), lambda i: (0, 0)),
        ],
        out_specs=pl.BlockSpec((c, br), lambda i: (0, i)),
        out_shape=jax.ShapeDtypeStruct((c, n), jnp.float32),
    )(accp2, degp, b2.reshape(1, h), Wlin, blin.reshape(1, c))

    return out.T
